# baseline (speedup 1.0000x reference)
import functools

import jax
import jax.numpy as jnp
from jax import lax
from jax.experimental import pallas as pl
from jax.experimental.pallas import tpu as pltpu
from jax.experimental.pallas import tpu_sc as plsc

_N = 10000
_E = 160000
_F = 256
_O = 512
_NC = 2
_NS = 16
_HALF = _F // _NC
_NQ = 4
_QW = _F // _NQ
_EPT = _E // _NS
_CHUNK = 100
_NCHUNK = _EPT // _CHUNK
_RPT = _N // _NS
_DEGW = 16


def _deg_body(ei_hbm, ones_hbm, zeros_hbm, deg_hbm, idx_v, ones_v, sem, acc):
    c = lax.axis_index("c")
    s = lax.axis_index("s")
    pltpu.sync_copy(zeros_hbm, acc.at[pl.ds(s * _RPT, _RPT)])
    pltpu.sync_copy(ones_hbm, ones_v)
    pltpu.sync_copy(ei_hbm.at[1 - c, s], idx_v)
    plsc.subcore_barrier()

    def step(j, carry):
        pltpu.async_copy(ones_v, acc.at[idx_v.at[j]], sem, add=True)
        return carry

    lax.fori_loop(0, _NCHUNK, step, 0)

    def drain(j, carry):
        pltpu.make_async_copy(ones_v, acc.at[idx_v.at[j]], sem).wait()
        return carry

    lax.fori_loop(0, _NCHUNK, drain, 0)
    plsc.subcore_barrier()
    pltpu.sync_copy(acc.at[pl.ds(s * _RPT, _RPT)], deg_hbm.at[c, s])


def _sc_degrees(ei4):
    ones = jnp.ones((_CHUNK, _DEGW), jnp.float32)
    zeros = jnp.zeros((_RPT, _DEGW), jnp.float32)
    mesh = plsc.VectorSubcoreMesh(core_axis_name="c", subcore_axis_name="s")
    f = pl.kernel(
        _deg_body,
        out_type=jax.ShapeDtypeStruct((_NC, _NS, _RPT, _DEGW), jnp.float32),
        mesh=mesh,
        scratch_types=[
            pltpu.VMEM((_NCHUNK, _CHUNK), jnp.int32),
            pltpu.VMEM((_CHUNK, _DEGW), jnp.float32),
            pltpu.SemaphoreType.DMA,
            pltpu.VMEM_SHARED((_N, _DEGW), jnp.float32),
        ],
    )
    return f(ei4, ones, zeros)


def _prescale_body(f_ref, dout_ref, h_ref):
    scale = jax.lax.rsqrt(dout_ref[...])
    x = f_ref[...] * scale
    for k in range(_NQ):
        h_ref[k, :, :] = x[:, k * _QW:(k + 1) * _QW]


def _tc_prescale(features, dout):
    blk = 1000
    grid = (_N // blk,)
    return pl.pallas_call(
        _prescale_body,
        grid=grid,
        in_specs=[
            pl.BlockSpec((blk, _F), lambda i: (i, 0)),
            pl.BlockSpec((blk, 1), lambda i: (i, 0)),
        ],
        out_specs=pl.BlockSpec((_NQ, blk, _QW), lambda i: (0, i, 0)),
        out_shape=jax.ShapeDtypeStruct((_NQ, _N, _QW), jnp.float32),
    )(features, dout)


def _agg_body(ei_hbm, h_hbm, zeros_hbm, agg_hbm,
              src_idx, dst_idx, buf0, buf1, gsem, ssem, acc):
    c = lax.axis_index("c")
    s = lax.axis_index("s")
    pltpu.sync_copy(ei_hbm.at[0, s], src_idx)
    pltpu.sync_copy(ei_hbm.at[1, s], dst_idx)

    bufs = (buf0, buf1)

    def gather(q, j, b):
        pltpu.async_copy(h_hbm.at[q].at[src_idx.at[j]], bufs[b], gsem)

    def wait_gather(q, j, b):
        pltpu.make_async_copy(h_hbm.at[q].at[src_idx.at[j]],
                              bufs[b], gsem).wait()

    def scatter(j, b):
        pltpu.async_copy(bufs[b], acc.at[dst_idx.at[j]], ssem, add=True)

    def wait_scatter(b):
        pltpu.make_async_copy(bufs[b], acc.at[dst_idx.at[0]], ssem).wait()

    for p in range(2):
        q = c * 2 + p

        def zinit(j, carry):
            pltpu.sync_copy(zeros_hbm, acc.at[pl.ds(s * _RPT + j * 125, 125)])
            return carry

        lax.fori_loop(0, _RPT // 125, zinit, 0)
        plsc.subcore_barrier()

        gather(q, 0, 0)

        def group(g, carry):
            for b in range(2):
                j = g * 2 + b
                wait_gather(q, j, b)

                @pl.when(j > 0)
                def _():
                    wait_scatter(1 - b)

                @pl.when(j < _NCHUNK - 1)
                def _():
                    gather(q, j + 1, 1 - b)

                scatter(j, b)
            return carry

        lax.fori_loop(0, _NCHUNK // 2, group, 0)
        wait_scatter(1)
        plsc.subcore_barrier()
        pltpu.sync_copy(acc.at[pl.ds(s * _RPT, _RPT)], agg_hbm.at[q, s])


def _sc_aggregate(ei4, h4):
    zeros = jnp.zeros((125, _QW), jnp.float32)
    mesh = plsc.VectorSubcoreMesh(core_axis_name="c", subcore_axis_name="s")
    f = pl.kernel(
        _agg_body,
        out_type=jax.ShapeDtypeStruct((_NQ, _NS, _RPT, _QW), jnp.float32),
        mesh=mesh,
        compiler_params=pltpu.CompilerParams(use_tc_tiling_on_sc=False),
        scratch_types=[
            pltpu.VMEM((_NCHUNK, _CHUNK), jnp.int32),
            pltpu.VMEM((_NCHUNK, _CHUNK), jnp.int32),
            pltpu.VMEM((_CHUNK, _QW), jnp.float32),
            pltpu.VMEM((_CHUNK, _QW), jnp.float32),
            pltpu.SemaphoreType.DMA,
            pltpu.SemaphoreType.DMA,
            pltpu.VMEM_SHARED((_N, _QW), jnp.float32),
        ],
    )
    return f(ei4, h4, zeros)


def _matmul_body(agg_ref, din_ref, wt_ref, b_ref, o_ref):
    x = jnp.concatenate([agg_ref[k] for k in range(_NQ)], axis=-1)
    x = x * jax.lax.rsqrt(din_ref[...])
    o_ref[...] = (jnp.dot(x, wt_ref[...],
                          preferred_element_type=jnp.float32)
                  + b_ref[...])


def _tc_matmul(agg4, din, W, b):
    blk = 1000
    grid = (_N // blk,)
    wt = W.T
    b2 = b.reshape(1, _O)
    return pl.pallas_call(
        _matmul_body,
        grid=grid,
        in_specs=[
            pl.BlockSpec((_NQ, blk, _QW), lambda i: (0, i, 0)),
            pl.BlockSpec((blk, 1), lambda i: (i, 0)),
            pl.BlockSpec((_F, _O), lambda i: (0, 0)),
            pl.BlockSpec((1, _O), lambda i: (0, 0)),
        ],
        out_specs=pl.BlockSpec((blk, _O), lambda i: (i, 0)),
        out_shape=jax.ShapeDtypeStruct((_N, _O), jnp.float32),
    )(agg4, din, wt, b2)


def kernel(features, edge_index, W, b):
    ei4 = edge_index.astype(jnp.int32).reshape(2, _NS, _NCHUNK, _CHUNK)
    deg = _sc_degrees(ei4).reshape(_NC, _N, _DEGW)
    din = deg[0, :, :1]
    dout = deg[1, :, :1]
    h4 = _tc_prescale(features, dout)
    agg4 = _sc_aggregate(ei4, h4).reshape(_NQ, _N, _QW)
    return _tc_matmul(agg4, din, W, b)


# speedup vs baseline: 1.3640x; 1.3640x over previous
import functools

import jax
import jax.numpy as jnp
from jax import lax
from jax.experimental import pallas as pl
from jax.experimental.pallas import tpu as pltpu
from jax.experimental.pallas import tpu_sc as plsc

_N = 10000
_E = 160000
_F = 256
_O = 512
_NC = 2
_NS = 16
_HALF = _F // _NC
_NQ = 4
_QW = _F // _NQ
_EPT = _E // _NS
_CHUNK = 80
_NCHUNK = _EPT // _CHUNK
_RPT = _N // _NS
_DEGW = 8


def _deg_body(ei_hbm, ones_hbm, zeros_hbm, deg_hbm, idx_v, ones_v, sem, acc):
    c = lax.axis_index("c")
    s = lax.axis_index("s")
    pltpu.sync_copy(zeros_hbm, acc.at[pl.ds(s * _RPT, _RPT)])
    pltpu.sync_copy(ones_hbm, ones_v)
    pltpu.sync_copy(ei_hbm.at[1 - c, s], idx_v)
    plsc.subcore_barrier()

    def step(j, carry):
        pltpu.sync_copy(ones_v, acc.at[idx_v.at[j]], add=True)
        return carry

    lax.fori_loop(0, _NCHUNK, step, 0)
    plsc.subcore_barrier()
    pltpu.sync_copy(acc.at[pl.ds(s * _RPT, _RPT)], deg_hbm.at[c, s])


def _sc_degrees(ei4):
    ones = jnp.ones((_CHUNK, _DEGW), jnp.float32)
    zeros = jnp.zeros((_RPT, _DEGW), jnp.float32)
    mesh = plsc.VectorSubcoreMesh(core_axis_name="c", subcore_axis_name="s")
    f = pl.kernel(
        _deg_body,
        out_type=jax.ShapeDtypeStruct((_NC, _NS, _RPT, _DEGW), jnp.float32),
        mesh=mesh,
        scratch_types=[
            pltpu.VMEM((_NCHUNK, _CHUNK), jnp.int32),
            pltpu.VMEM((_CHUNK, _DEGW), jnp.float32),
            pltpu.SemaphoreType.DMA,
            pltpu.VMEM_SHARED((_N, _DEGW), jnp.float32),
        ],
    )
    return f(ei4, ones, zeros)


def _prescale_body(f_ref, dout_ref, h_ref):
    scale = jax.lax.rsqrt(dout_ref[...])
    x = f_ref[...] * scale
    h_ref[0, :, :] = x[:, :_HALF]
    h_ref[1, :, :] = x[:, _HALF:]


def _tc_prescale(features, dout):
    blk = 1000
    grid = (_N // blk,)
    return pl.pallas_call(
        _prescale_body,
        grid=grid,
        in_specs=[
            pl.BlockSpec((blk, _F), lambda i: (i, 0)),
            pl.BlockSpec((blk, 1), lambda i: (i, 0)),
        ],
        out_specs=pl.BlockSpec((_NC, blk, _HALF), lambda i: (0, i, 0)),
        out_shape=jax.ShapeDtypeStruct((_NC, _N, _HALF), jnp.float32),
    )(features, dout)


def _agg_body(eis_hbm, eid_hbm, h_hbm, zeros_hbm, agg_hbm,
              src_idx, dst_idx, buf0, buf1, gsem, acc):
    c = lax.axis_index("c")
    s = lax.axis_index("s")
    pltpu.sync_copy(eis_hbm.at[s], src_idx)
    pltpu.sync_copy(eid_hbm.at[s], dst_idx)

    bufs = (buf0, buf1)

    def gather(q, j, b):
        pltpu.async_copy(h_hbm.at[q].at[src_idx.at[pl.ds(j * _CHUNK, _CHUNK)]],
                         bufs[b], gsem)

    def wait_gather(q, j, b):
        pltpu.make_async_copy(
            h_hbm.at[q].at[src_idx.at[pl.ds(j * _CHUNK, _CHUNK)]],
            bufs[b], gsem).wait()

    def scatter(j, b):
        pltpu.sync_copy(bufs[b], acc.at[dst_idx.at[j]], add=True)

    def zinit(j, carry):
        pltpu.sync_copy(zeros_hbm, acc.at[pl.ds(s * _RPT + j * 125, 125)])
        return carry

    lax.fori_loop(0, _RPT // 125, zinit, 0)
    plsc.subcore_barrier()

    def group(g, carry):
        gather(c, 2 * g, 0)
        gather(c, 2 * g + 1, 1)
        wait_gather(c, 2 * g, 0)
        scatter(2 * g, 0)
        wait_gather(c, 2 * g + 1, 1)
        scatter(2 * g + 1, 1)
        return carry

    lax.fori_loop(0, _NCHUNK // 2, group, 0)
    plsc.subcore_barrier()
    pltpu.sync_copy(acc.at[pl.ds(s * _RPT, _RPT)], agg_hbm.at[c, s])


def _sc_aggregate(eis, eid, h3):
    zeros = jnp.zeros((125, _HALF), jnp.float32)
    mesh = plsc.VectorSubcoreMesh(core_axis_name="c", subcore_axis_name="s")
    f = pl.kernel(
        _agg_body,
        out_type=jax.ShapeDtypeStruct((_NC, _NS, _RPT, _HALF), jnp.float32),
        mesh=mesh,
        scratch_types=[
            pltpu.VMEM((_EPT,), jnp.int32),
            pltpu.VMEM((_NCHUNK, _CHUNK), jnp.int32),
            pltpu.VMEM((_CHUNK, _HALF), jnp.float32),
            pltpu.VMEM((_CHUNK, _HALF), jnp.float32),
            pltpu.SemaphoreType.DMA,
            pltpu.VMEM_SHARED((_N, _HALF), jnp.float32),
        ],
    )
    return f(eis, eid, h3, zeros)


def _matmul_body(agg_ref, din_ref, wt_ref, b_ref, o_ref):
    x = jnp.concatenate([agg_ref[0], agg_ref[1]], axis=-1)
    x = x * jax.lax.rsqrt(din_ref[...])
    o_ref[...] = (jnp.dot(x, wt_ref[...],
                          preferred_element_type=jnp.float32)
                  + b_ref[...])


def _tc_matmul(agg3, din, W, b):
    blk = 1000
    grid = (_N // blk,)
    wt = W.T
    b2 = b.reshape(1, _O)
    return pl.pallas_call(
        _matmul_body,
        grid=grid,
        in_specs=[
            pl.BlockSpec((_NC, blk, _HALF), lambda i: (0, i, 0)),
            pl.BlockSpec((blk, 1), lambda i: (i, 0)),
            pl.BlockSpec((_F, _O), lambda i: (0, 0)),
            pl.BlockSpec((1, _O), lambda i: (0, 0)),
        ],
        out_specs=pl.BlockSpec((blk, _O), lambda i: (i, 0)),
        out_shape=jax.ShapeDtypeStruct((_N, _O), jnp.float32),
    )(agg3, din, wt, b2)


def kernel(features, edge_index, W, b):
    ei32 = edge_index.astype(jnp.int32)
    ei4 = ei32.reshape(2, _NS, _NCHUNK, _CHUNK)
    eis = ei32[0].reshape(_NS, _EPT)
    eid = ei32[1].reshape(_NS, _NCHUNK, _CHUNK)
    deg = _sc_degrees(ei4).reshape(_NC, _N, _DEGW)
    din = deg[0, :, :1]
    dout = deg[1, :, :1]
    h3 = _tc_prescale(features, dout)
    agg3 = _sc_aggregate(eis, eid, h3).reshape(_NC, _N, _HALF)
    return _tc_matmul(agg3, din, W, b)


# speedup vs baseline: 1.3874x; 1.0172x over previous
import functools

import jax
import jax.numpy as jnp
from jax import lax
from jax.experimental import pallas as pl
from jax.experimental.pallas import tpu as pltpu
from jax.experimental.pallas import tpu_sc as plsc

_N = 10000
_E = 160000
_F = 256
_O = 512
_NC = 2
_NS = 16
_HALF = _F // _NC
_NQ = 4
_QW = _F // _NQ
_EPT = _E // _NS
_CHUNK = 80
_NCHUNK = _EPT // _CHUNK
_RPT = _N // _NS
_DEGW = 8


def _deg_body(ei_hbm, ones_hbm, zeros_hbm, deg_hbm, idx_v, ones_v, sem, acc):
    c = lax.axis_index("c")
    s = lax.axis_index("s")
    pltpu.sync_copy(zeros_hbm, acc.at[pl.ds(s * _RPT, _RPT)])
    pltpu.sync_copy(ones_hbm, ones_v)
    pltpu.sync_copy(ei_hbm.at[1 - c, s], idx_v)
    plsc.subcore_barrier()

    def step(g, carry):
        pltpu.async_copy(ones_v, acc.at[idx_v.at[2 * g]], sem, add=True)
        pltpu.async_copy(ones_v, acc.at[idx_v.at[2 * g + 1]], sem, add=True)
        pltpu.make_async_copy(ones_v, acc.at[idx_v.at[2 * g]], sem).wait()
        pltpu.make_async_copy(ones_v, acc.at[idx_v.at[2 * g + 1]], sem).wait()
        return carry

    lax.fori_loop(0, _NCHUNK // 2, step, 0)
    plsc.subcore_barrier()
    pltpu.sync_copy(acc.at[pl.ds(s * _RPT, _RPT)], deg_hbm.at[c, s])


def _sc_degrees(ei4):
    ones = jnp.ones((_CHUNK, _DEGW), jnp.float32)
    zeros = jnp.zeros((_RPT, _DEGW), jnp.float32)
    mesh = plsc.VectorSubcoreMesh(core_axis_name="c", subcore_axis_name="s")
    f = pl.kernel(
        _deg_body,
        out_type=jax.ShapeDtypeStruct((_NC, _NS, _RPT, _DEGW), jnp.float32),
        mesh=mesh,
        scratch_types=[
            pltpu.VMEM((_NCHUNK, _CHUNK), jnp.int32),
            pltpu.VMEM((_CHUNK, _DEGW), jnp.float32),
            pltpu.SemaphoreType.DMA,
            pltpu.VMEM_SHARED((_N, _DEGW), jnp.float32),
        ],
    )
    return f(ei4, ones, zeros)


def _prescale_body(f_ref, dout_ref, h_ref):
    scale = jax.lax.rsqrt(dout_ref[...])
    x = f_ref[...] * scale
    h_ref[0, :, :] = x[:, :_HALF]
    h_ref[1, :, :] = x[:, _HALF:]


def _tc_prescale(features, dout):
    blk = 1000
    grid = (_N // blk,)
    return pl.pallas_call(
        _prescale_body,
        grid=grid,
        in_specs=[
            pl.BlockSpec((blk, _F), lambda i: (i, 0)),
            pl.BlockSpec((blk, 1), lambda i: (i, 0)),
        ],
        out_specs=pl.BlockSpec((_NC, blk, _HALF), lambda i: (0, i, 0)),
        out_shape=jax.ShapeDtypeStruct((_NC, _N, _HALF), jnp.float32),
    )(features, dout)


def _agg_body(eis_hbm, eid_hbm, h_hbm, zeros_hbm, agg_hbm,
              src_idx, dst_idx, buf0, buf1, gsem, acc):
    c = lax.axis_index("c")
    s = lax.axis_index("s")
    pltpu.sync_copy(eis_hbm.at[s], src_idx)
    pltpu.sync_copy(eid_hbm.at[s], dst_idx)

    bufs = (buf0, buf1)

    def gather(q, j, b):
        pltpu.async_copy(h_hbm.at[q].at[src_idx.at[pl.ds(j * _CHUNK, _CHUNK)]],
                         bufs[b], gsem)

    def wait_gather(q, j, b):
        pltpu.make_async_copy(
            h_hbm.at[q].at[src_idx.at[pl.ds(j * _CHUNK, _CHUNK)]],
            bufs[b], gsem).wait()

    def scatter(j, b):
        pltpu.sync_copy(bufs[b], acc.at[dst_idx.at[j]], add=True)

    def zinit(j, carry):
        pltpu.sync_copy(zeros_hbm, acc.at[pl.ds(s * _RPT + j * 125, 125)])
        return carry

    lax.fori_loop(0, _RPT // 125, zinit, 0)
    plsc.subcore_barrier()

    def group(g, carry):
        gather(c, 2 * g, 0)
        gather(c, 2 * g + 1, 1)
        wait_gather(c, 2 * g, 0)
        scatter(2 * g, 0)
        wait_gather(c, 2 * g + 1, 1)
        scatter(2 * g + 1, 1)
        return carry

    lax.fori_loop(0, _NCHUNK // 2, group, 0)
    plsc.subcore_barrier()
    pltpu.sync_copy(acc.at[pl.ds(s * _RPT, _RPT)], agg_hbm.at[c, s])


def _sc_aggregate(eis, eid, h3):
    zeros = jnp.zeros((125, _HALF), jnp.float32)
    mesh = plsc.VectorSubcoreMesh(core_axis_name="c", subcore_axis_name="s")
    f = pl.kernel(
        _agg_body,
        out_type=jax.ShapeDtypeStruct((_NC, _NS, _RPT, _HALF), jnp.float32),
        mesh=mesh,
        scratch_types=[
            pltpu.VMEM((_EPT,), jnp.int32),
            pltpu.VMEM((_NCHUNK, _CHUNK), jnp.int32),
            pltpu.VMEM((_CHUNK, _HALF), jnp.float32),
            pltpu.VMEM((_CHUNK, _HALF), jnp.float32),
            pltpu.SemaphoreType.DMA,
            pltpu.VMEM_SHARED((_N, _HALF), jnp.float32),
        ],
    )
    return f(eis, eid, h3, zeros)


def _matmul_body(agg_ref, din_ref, wt_ref, b_ref, o_ref):
    x = jnp.concatenate([agg_ref[0], agg_ref[1]], axis=-1)
    x = x * jax.lax.rsqrt(din_ref[...])
    o_ref[...] = (jnp.dot(x, wt_ref[...],
                          preferred_element_type=jnp.float32)
                  + b_ref[...])


def _tc_matmul(agg3, din, W, b):
    blk = 1000
    grid = (_N // blk,)
    wt = W.T
    b2 = b.reshape(1, _O)
    return pl.pallas_call(
        _matmul_body,
        grid=grid,
        in_specs=[
            pl.BlockSpec((_NC, blk, _HALF), lambda i: (0, i, 0)),
            pl.BlockSpec((blk, 1), lambda i: (i, 0)),
            pl.BlockSpec((_F, _O), lambda i: (0, 0)),
            pl.BlockSpec((1, _O), lambda i: (0, 0)),
        ],
        out_specs=pl.BlockSpec((blk, _O), lambda i: (i, 0)),
        out_shape=jax.ShapeDtypeStruct((_N, _O), jnp.float32),
    )(agg3, din, wt, b2)


def kernel(features, edge_index, W, b):
    ei32 = edge_index.astype(jnp.int32)
    ei4 = ei32.reshape(2, _NS, _NCHUNK, _CHUNK)
    eis = ei32[0].reshape(_NS, _EPT)
    eid = ei32[1].reshape(_NS, _NCHUNK, _CHUNK)
    deg = _sc_degrees(ei4).reshape(_NC, _N, _DEGW)
    din = deg[0, :, :1]
    dout = deg[1, :, :1]
    h3 = _tc_prescale(features, dout)
    agg3 = _sc_aggregate(eis, eid, h3).reshape(_NC, _N, _HALF)
    return _tc_matmul(agg3, din, W, b)


# speedup vs baseline: 1.3908x; 1.0025x over previous
import functools

import jax
import jax.numpy as jnp
from jax import lax
from jax.experimental import pallas as pl
from jax.experimental.pallas import tpu as pltpu
from jax.experimental.pallas import tpu_sc as plsc

_N = 10000
_E = 160000
_F = 256
_O = 512
_NC = 2
_NS = 16
_HALF = _F // _NC
_NQ = 4
_QW = _F // _NQ
_EPT = _E // _NS
_CHUNK = 80
_NCHUNK = _EPT // _CHUNK
_RPT = _N // _NS
_DEGW = 8


def _deg_body(ei_hbm, ones_hbm, zeros_hbm, deg_hbm, idx_v, ones_v, sem, acc):
    c = lax.axis_index("c")
    s = lax.axis_index("s")
    pltpu.sync_copy(zeros_hbm, acc.at[pl.ds(s * _RPT, _RPT)])
    pltpu.sync_copy(ones_hbm, ones_v)
    pltpu.sync_copy(ei_hbm.at[1 - c, s], idx_v)
    plsc.subcore_barrier()

    def step(g, carry):
        pltpu.async_copy(ones_v, acc.at[idx_v.at[2 * g]], sem, add=True)
        pltpu.async_copy(ones_v, acc.at[idx_v.at[2 * g + 1]], sem, add=True)
        pltpu.make_async_copy(ones_v, acc.at[idx_v.at[2 * g]], sem).wait()
        pltpu.make_async_copy(ones_v, acc.at[idx_v.at[2 * g + 1]], sem).wait()
        return carry

    lax.fori_loop(0, _NCHUNK // 2, step, 0)
    if _NCHUNK % 2:
        pltpu.sync_copy(ones_v, acc.at[idx_v.at[_NCHUNK - 1]], add=True)
    plsc.subcore_barrier()
    pltpu.sync_copy(acc.at[pl.ds(s * _RPT, _RPT)], deg_hbm.at[c, s])


def _sc_degrees(ei4):
    ones = jnp.ones((_CHUNK, _DEGW), jnp.float32)
    zeros = jnp.zeros((_RPT, _DEGW), jnp.float32)
    mesh = plsc.VectorSubcoreMesh(core_axis_name="c", subcore_axis_name="s")
    f = pl.kernel(
        _deg_body,
        out_type=jax.ShapeDtypeStruct((_NC, _NS, _RPT, _DEGW), jnp.float32),
        mesh=mesh,
        compiler_params=pltpu.CompilerParams(use_tc_tiling_on_sc=False),
        scratch_types=[
            pltpu.VMEM((_NCHUNK, _CHUNK), jnp.int32),
            pltpu.VMEM((_CHUNK, _DEGW), jnp.float32),
            pltpu.SemaphoreType.DMA,
            pltpu.VMEM_SHARED((_N, _DEGW), jnp.float32),
        ],
    )
    return f(ei4, ones, zeros)


def _prescale_body(f_ref, dout_ref, h_ref):
    scale = jax.lax.rsqrt(dout_ref[...])
    x = f_ref[...] * scale
    h_ref[0, :, :] = x[:, :_HALF]
    h_ref[1, :, :] = x[:, _HALF:]


def _tc_prescale(features, dout):
    blk = 1000
    grid = (_N // blk,)
    return pl.pallas_call(
        _prescale_body,
        grid=grid,
        in_specs=[
            pl.BlockSpec((blk, _F), lambda i: (i, 0)),
            pl.BlockSpec((blk, 1), lambda i: (i, 0)),
        ],
        out_specs=pl.BlockSpec((_NC, blk, _HALF), lambda i: (0, i, 0)),
        out_shape=jax.ShapeDtypeStruct((_NC, _N, _HALF), jnp.float32),
    )(features, dout)


def _agg_body(eis_hbm, eid_hbm, h_hbm, zeros_hbm, agg_hbm,
              src_idx, dst_idx, buf0, buf1, gsem, acc):
    c = lax.axis_index("c")
    s = lax.axis_index("s")
    pltpu.sync_copy(eis_hbm.at[s], src_idx)
    pltpu.sync_copy(eid_hbm.at[s], dst_idx)

    bufs = (buf0, buf1)

    def gather(q, j, b):
        pltpu.async_copy(h_hbm.at[q].at[src_idx.at[pl.ds(j * _CHUNK, _CHUNK)]],
                         bufs[b], gsem)

    def wait_gather(q, j, b):
        pltpu.make_async_copy(
            h_hbm.at[q].at[src_idx.at[pl.ds(j * _CHUNK, _CHUNK)]],
            bufs[b], gsem).wait()

    def scatter(j, b):
        pltpu.sync_copy(bufs[b], acc.at[dst_idx.at[j]], add=True)

    def zinit(j, carry):
        pltpu.sync_copy(zeros_hbm, acc.at[pl.ds(s * _RPT + j * 125, 125)])
        return carry

    lax.fori_loop(0, _RPT // 125, zinit, 0)
    plsc.subcore_barrier()

    def group(g, carry):
        gather(c, 2 * g, 0)
        gather(c, 2 * g + 1, 1)
        wait_gather(c, 2 * g, 0)
        scatter(2 * g, 0)
        wait_gather(c, 2 * g + 1, 1)
        scatter(2 * g + 1, 1)
        return carry

    lax.fori_loop(0, _NCHUNK // 2, group, 0)
    if _NCHUNK % 2:
        gather(c, _NCHUNK - 1, 0)
        wait_gather(c, _NCHUNK - 1, 0)
        scatter(_NCHUNK - 1, 0)
    plsc.subcore_barrier()
    pltpu.sync_copy(acc.at[pl.ds(s * _RPT, _RPT)], agg_hbm.at[c, s])


def _sc_aggregate(eis, eid, h3):
    zeros = jnp.zeros((125, _HALF), jnp.float32)
    mesh = plsc.VectorSubcoreMesh(core_axis_name="c", subcore_axis_name="s")
    f = pl.kernel(
        _agg_body,
        out_type=jax.ShapeDtypeStruct((_NC, _NS, _RPT, _HALF), jnp.float32),
        mesh=mesh,
        scratch_types=[
            pltpu.VMEM((_EPT,), jnp.int32),
            pltpu.VMEM((_NCHUNK, _CHUNK), jnp.int32),
            pltpu.VMEM((_CHUNK, _HALF), jnp.float32),
            pltpu.VMEM((_CHUNK, _HALF), jnp.float32),
            pltpu.SemaphoreType.DMA,
            pltpu.VMEM_SHARED((_N, _HALF), jnp.float32),
        ],
    )
    return f(eis, eid, h3, zeros)


def _matmul_body(agg_ref, din_ref, wt_ref, b_ref, o_ref):
    x = jnp.concatenate([agg_ref[0], agg_ref[1]], axis=-1)
    x = x * jax.lax.rsqrt(din_ref[...])
    o_ref[...] = (jnp.dot(x, wt_ref[...],
                          preferred_element_type=jnp.float32)
                  + b_ref[...])


def _tc_matmul(agg3, din, W, b):
    blk = 1000
    grid = (_N // blk,)
    wt = W.T
    b2 = b.reshape(1, _O)
    return pl.pallas_call(
        _matmul_body,
        grid=grid,
        in_specs=[
            pl.BlockSpec((_NC, blk, _HALF), lambda i: (0, i, 0)),
            pl.BlockSpec((blk, 1), lambda i: (i, 0)),
            pl.BlockSpec((_F, _O), lambda i: (0, 0)),
            pl.BlockSpec((1, _O), lambda i: (0, 0)),
        ],
        out_specs=pl.BlockSpec((blk, _O), lambda i: (i, 0)),
        out_shape=jax.ShapeDtypeStruct((_N, _O), jnp.float32),
    )(agg3, din, wt, b2)


def kernel(features, edge_index, W, b):
    ei32 = edge_index.astype(jnp.int32)
    ei4 = ei32.reshape(2, _NS, _NCHUNK, _CHUNK)
    eis = ei32[0].reshape(_NS, _EPT)
    eid = ei32[1].reshape(_NS, _NCHUNK, _CHUNK)
    deg = _sc_degrees(ei4).reshape(_NC, _N, _DEGW)
    din = deg[0, :, :1]
    dout = deg[1, :, :1]
    h3 = _tc_prescale(features, dout)
    agg3 = _sc_aggregate(eis, eid, h3).reshape(_NC, _N, _HALF)
    return _tc_matmul(agg3, din, W, b)
